# Initial kernel scaffold; baseline (speedup 1.0000x reference)
import jax
import jax.numpy as jnp
from jax.experimental import pallas as pl


def kernel(x, edge_index, h, node_graph_ids, gW1, gb1, gW2, gb2, gW3, gb3, Wa, ba, Wi, bi, Wh, bh, Wmu, bmu, Wsig, bsig):
    raise NotImplementedError("write your pallas kernel here")


# speedup vs baseline: 6.1421x; 6.1421x over previous
import functools

import jax
import jax.numpy as jnp
from jax import lax
from jax.experimental import pallas as pl
from jax.experimental.pallas import tpu as pltpu
from jax.experimental.pallas import tpu_sc as plsc

N = 10000
E = 320000
G = 100
FP = 128
GP = 128
HALF = N // 2
SLAB = HALF + 8
TRASH = HALF
EPT = E // 16
CH = 125
NCH = EPT // CH
ROWS_PT = 1000


NH = 10240
NHH = NH // 2


def _sc_degrees(sd_idx):
    mesh = plsc.VectorSubcoreMesh(core_axis_name="c", subcore_axis_name="s")

    @functools.partial(
        pl.kernel,
        mesh=mesh,
        out_type=jax.ShapeDtypeStruct((2, 80, 128), jnp.float32),
        compiler_params=pltpu.CompilerParams(needs_layout_passes=False),
        scratch_types=[
            pltpu.VMEM((20000,), jnp.int32),
            pltpu.VMEM((16 * NHH,), jnp.float32),
            pltpu.VMEM((80, 128), jnp.float32),
            pltpu.VMEM((8, 128), jnp.float32),
            pltpu.VMEM((8, 128), jnp.float32),
            pltpu.VMEM_SHARED((16, 80, 128), jnp.float32),
        ],
    )
    def k(sd_hbm, z1_hbm, out_hbm, idx_v, hist_v, red_v, tmp_v, acc_v, hists_sh):
        c = lax.axis_index("c")
        s = lax.axis_index("s")
        pltpu.sync_copy(sd_hbm.at[c, s], idx_v)
        lanes = lax.broadcasted_iota(jnp.int32, (16,), 0)
        ones16 = jnp.ones((16,), jnp.float32)

        for p in range(2):
            base = p * NHH
            pltpu.sync_copy(z1_hbm, hist_v)

            def sbody(i, carry):
                v = idx_v[pl.ds(i * 16, 16)]
                rel = v - base
                m = (rel >= 0) & (rel < NHH)
                relc = jnp.clip(rel, 0, NHH - 1)
                addr = relc + lanes * NHH
                plsc.addupdate_scatter(hist_v, [addr], ones16, mask=m)
                return carry

            lax.fori_loop(0, 1250, sbody, 0)

            def mbody(r, carry):
                for q in range(8):
                    acc = hist_v[pl.ds(r * 128 + q * 16, 16)]
                    for l in range(1, 16):
                        acc = acc + hist_v[pl.ds(l * NHH + r * 128 + q * 16, 16)]
                    red_v[p * 40 + r, pl.ds(q * 16, 16)] = acc
                return carry

            lax.fori_loop(0, 40, mbody, 0)

        pltpu.sync_copy(red_v, hists_sh.at[s])
        plsc.subcore_barrier()

        @pl.when(s < 10)
        def _():
            pltpu.sync_copy(hists_sh.at[0, pl.ds(s * 8, 8)], acc_v)

            def rbody(t, carry):
                pltpu.sync_copy(hists_sh.at[t, pl.ds(s * 8, 8)], tmp_v)
                for r in range(8):
                    for q in range(8):
                        acc_v[r, pl.ds(q * 16, 16)] = (
                            acc_v[r, pl.ds(q * 16, 16)]
                            + tmp_v[r, pl.ds(q * 16, 16)])
                return carry

            lax.fori_loop(1, 16, rbody, 0)
            pltpu.sync_copy(acc_v, out_hbm.at[c, pl.ds(s * 8, 8)])

    return k(sd_idx, jnp.zeros((16 * NHH,), jnp.float32))


def _sc_edge_agg(val, src_idx, dst_c, zeros_rows):
    mesh = plsc.VectorSubcoreMesh(core_axis_name="c", subcore_axis_name="s")

    @functools.partial(
        pl.kernel,
        mesh=mesh,
        out_type=jax.ShapeDtypeStruct((2, SLAB, FP), jnp.float32),
        scratch_types=[
            pltpu.VMEM((NCH, CH), jnp.int32),
            pltpu.VMEM((NCH, CH), jnp.int32),
            pltpu.VMEM((CH, FP), jnp.float32),
            pltpu.VMEM((CH, FP), jnp.float32),
            pltpu.VMEM_SHARED((SLAB, FP), jnp.float32),
            pltpu.SemaphoreType.DMA,
            pltpu.SemaphoreType.DMA,
        ],
    )
    def k(val_hbm, src_hbm, dst_hbm, z_hbm, out_hbm,
          sidx, didx, buf0, buf1, agg_sh, sem0, sem1):
        c = lax.axis_index("c")
        s = lax.axis_index("s")
        pltpu.sync_copy(src_hbm.at[s], sidx)
        pltpu.sync_copy(dst_hbm.at[c, s], didx)

        @pl.when(s < 5)
        def _():
            pltpu.sync_copy(z_hbm, agg_sh.at[pl.ds(s * ROWS_PT, ROWS_PT)])

        @pl.when(s == 5)
        def _():
            pltpu.sync_copy(z_hbm.at[pl.ds(0, 8)],
                            agg_sh.at[pl.ds(HALF, 8)])

        plsc.subcore_barrier()

        bufs = (buf0, buf1)
        sems = (sem0, sem1)
        pltpu.async_copy(val_hbm.at[sidx.at[0]], buf0, sem0)

        def body(g, carry):
            for b in range(2):
                j = g * 2 + b
                buf = bufs[b]
                sem = sems[b]
                nbuf = bufs[1 - b]
                nsem = sems[1 - b]

                @pl.when(j < NCH - 1)
                def _():
                    pltpu.async_copy(val_hbm.at[sidx.at[j + 1]], nbuf, nsem)

                pltpu.make_async_copy(val_hbm.at[sidx.at[j]], buf, sem).wait()
                pltpu.sync_copy(buf, agg_sh.at[didx.at[j]], add=True)
            return carry

        lax.fori_loop(0, NCH // 2, body, 0)
        plsc.subcore_barrier()

        @pl.when(s < 5)
        def _():
            pltpu.sync_copy(
                agg_sh.at[pl.ds(s * ROWS_PT, ROWS_PT)],
                out_hbm.at[c, pl.ds(s * ROWS_PT, ROWS_PT)],
            )

    return k(val, src_idx, dst_c, zeros_rows)


def _norm(d):
    return lax.rsqrt(jnp.where(d > 0.0, d, 1.0))


def _dot16(a, b):
    return jnp.dot(a.astype(jnp.bfloat16), b.astype(jnp.bfloat16),
                   preferred_element_type=jnp.float32)


def _agg_spec():
    return pl.BlockSpec((1, 1000, FP), lambda i: (i // 5, i % 5, 0))


def _tc_pre(h, deg_out, w1p):
    def body(h_ref, d_ref, w_ref, o_ref):
        ns = _norm(d_ref[...])
        o_ref[...] = _dot16(h_ref[...] * ns, w_ref[...])

    return pl.pallas_call(
        body,
        grid=(10,),
        in_specs=[
            pl.BlockSpec((1000, 128), lambda i: (i, 0)),
            pl.BlockSpec((1000, 1), lambda i: (i, 0)),
            pl.BlockSpec((128, FP), lambda i: (0, 0)),
        ],
        out_specs=pl.BlockSpec((1000, FP), lambda i: (i, 0)),
        out_shape=jax.ShapeDtypeStruct((N, FP), jnp.float32),
    )(h, deg_out, w1p)


def _tc_mid1(agg, deg_in, deg_out, b1p):
    def body(a_ref, di_ref, do_ref, b_ref, o_ref):
        a = a_ref[0]
        nd = _norm(di_ref[...])
        ns = _norm(do_ref[...])
        o_ref[...] = jnp.maximum(a * nd + b_ref[...], 0.0) * ns

    return pl.pallas_call(
        body,
        grid=(10,),
        in_specs=[
            _agg_spec(),
            pl.BlockSpec((1000, 1), lambda i: (i, 0)),
            pl.BlockSpec((1000, 1), lambda i: (i, 0)),
            pl.BlockSpec((1, FP), lambda i: (0, 0)),
        ],
        out_specs=pl.BlockSpec((1000, FP), lambda i: (i, 0)),
        out_shape=jax.ShapeDtypeStruct((N, FP), jnp.float32),
    )(agg, deg_in, deg_out, b1p)


def _tc_mid2(agg, deg_in, deg_out, w2p, b2p):
    def body(a_ref, di_ref, do_ref, w_ref, b_ref, o_ref):
        a = a_ref[0]
        nd = _norm(di_ref[...])
        ns = _norm(do_ref[...])
        m = _dot16(a, w_ref[...])
        o_ref[...] = jnp.maximum(m * nd + b_ref[...], 0.0) * ns

    return pl.pallas_call(
        body,
        grid=(10,),
        in_specs=[
            _agg_spec(),
            pl.BlockSpec((1000, 1), lambda i: (i, 0)),
            pl.BlockSpec((1000, 1), lambda i: (i, 0)),
            pl.BlockSpec((FP, FP), lambda i: (0, 0)),
            pl.BlockSpec((1, FP), lambda i: (0, 0)),
        ],
        out_specs=pl.BlockSpec((1000, FP), lambda i: (i, 0)),
        out_shape=jax.ShapeDtypeStruct((N, FP), jnp.float32),
    )(agg, deg_in, deg_out, w2p, b2p)


def _tc_head(agg, deg_in, gid_row, w3p, b3, wa, ba, xp,
             wi, bi, wh, bh, wmu, bmu, wsig, bsig):
    def body(a_ref, di_ref, g_ref, w3_ref, b3_ref, wa_ref, ba_ref, x_ref,
             wi_ref, bi_ref, wh_ref, bh_ref, wmu_ref, bmu_ref,
             wsig_ref, bsig_ref, mu_ref, sig_ref, pooled):
        i = pl.program_id(0)

        @pl.when(i == 0)
        def _():
            pooled[...] = jnp.zeros_like(pooled)

        a = a_ref[0]
        nd = _norm(di_ref[...])
        z3 = _dot16(a, w3_ref[...]) * nd + b3_ref[...]
        gid = g_ref[0]
        gi = lax.broadcasted_iota(jnp.int32, (GP, 1), 0)
        mask = (gi == gid).astype(jnp.float32)
        pooled[...] += jnp.dot(mask, z3, preferred_element_type=jnp.float32,
                               precision=jax.lax.Precision.HIGHEST)

        @pl.when(i == 9)
        def _():
            xg = _dot16(pooled[...], wa_ref[...]) + ba_ref[...]
            u = jnp.concatenate([x_ref[...], xg], axis=1)
            u = jnp.maximum(_dot16(u, wi_ref[...]) + bi_ref[...], 0.0)
            u = jnp.maximum(_dot16(u, wh_ref[...]) + bh_ref[...], 0.0)
            for wref, bref, oref in ((wmu_ref, bmu_ref, mu_ref),
                                     (wsig_ref, bsig_ref, sig_ref)):
                v = _dot16(u, wref[...]) + bref[...]
                vv = v[:G]
                m = jnp.mean(vv, axis=0, keepdims=True)
                var = jnp.mean((vv - m) ** 2, axis=0, keepdims=True)
                oref[...] = (v - m) * lax.rsqrt(var + 1e-5)

    full = lambda shape: pl.BlockSpec(shape, lambda i: tuple(0 for _ in shape))
    return pl.pallas_call(
        body,
        grid=(10,),
        in_specs=[
            _agg_spec(),
            pl.BlockSpec((1000, 1), lambda i: (i, 0)),
            pl.BlockSpec((1, 1, 1000), lambda i: (i, 0, 0)),
            full((FP, 768)),
            full((1, 768)),
            full((768, 256)),
            full((1, 256)),
            full((GP, 256)),
            full((512, 512)),
            full((1, 512)),
            full((512, 512)),
            full((1, 512)),
            full((512, 64)),
            full((1, 64)),
            full((512, 64)),
            full((1, 64)),
        ],
        out_specs=[full((GP, 64)), full((GP, 64))],
        out_shape=[jax.ShapeDtypeStruct((GP, 64), jnp.float32),
                   jax.ShapeDtypeStruct((GP, 64), jnp.float32)],
        scratch_shapes=[pltpu.VMEM((GP, 768), jnp.float32)],
    )(agg, deg_in, gid_row, w3p, b3, wa, ba, xp,
      wi, bi, wh, bh, wmu, bmu, wsig, bsig)


def kernel(x, edge_index, h, node_graph_ids, gW1, gb1, gW2, gb2, gW3, gb3,
           Wa, ba, Wi, bi, Wh, bh, Wmu, bmu, Wsig, bsig):
    src, dst = edge_index[0], edge_index[1]
    src_t = src.reshape(16, NCH, CH)
    sd_idx = edge_index.reshape(2, 16, EPT)
    dst_lo = jnp.where(dst < HALF, dst, TRASH)
    dst_hi = jnp.where(dst >= HALF, dst - HALF, TRASH)
    dst_c = jnp.stack([dst_lo, dst_hi]).reshape(2, 16, NCH, CH)

    zeros_rows = jnp.zeros((ROWS_PT, FP), jnp.float32)

    degs = _sc_degrees(sd_idx).reshape(2, NH)
    deg_out = degs[0, :N].reshape(N, 1)
    deg_in = degs[1, :N].reshape(N, 1)

    pad_f = FP - gW1.shape[1]
    w1p = jnp.pad(gW1, ((0, 0), (0, pad_f)))
    b1p = jnp.pad(gb1, (0, pad_f)).reshape(1, FP)
    w2p = jnp.pad(gW2, ((0, pad_f), (0, pad_f)))
    b2p = jnp.pad(gb2, (0, pad_f)).reshape(1, FP)
    w3p = jnp.pad(gW3, ((0, pad_f), (0, 0)))
    xp = jnp.pad(x, ((0, GP - G), (0, 0)))
    gid_row = node_graph_ids.reshape(10, 1, 1000)

    h1 = _tc_pre(h, deg_out, w1p)
    agg1 = _sc_edge_agg(h1, src_t, dst_c, zeros_rows)
    z1n = _tc_mid1(agg1, deg_in, deg_out, b1p)
    agg2 = _sc_edge_agg(z1n, src_t, dst_c, zeros_rows)
    z2n = _tc_mid2(agg2, deg_in, deg_out, w2p, b2p)
    agg3 = _sc_edge_agg(z2n, src_t, dst_c, zeros_rows)
    mu, sig = _tc_head(agg3, deg_in, gid_row, w3p, gb3.reshape(1, 768),
                       Wa, ba.reshape(1, 256), xp,
                       Wi, bi.reshape(1, 512), Wh, bh.reshape(1, 512),
                       Wmu, bmu.reshape(1, 64), Wsig, bsig.reshape(1, 64))
    return mu[:G], sig[:G]
